# speedup vs baseline: 5.6887x; 5.6887x over previous
import functools

import jax
import jax.numpy as jnp
from jax import lax
from jax.experimental import pallas as pl
from jax.experimental.pallas import tpu as pltpu
from jax.experimental.pallas import tpu_sc as plsc

HIDDEN = 128
EPS = 1e-12

NC, NS = 2, 16
NW = NC * NS
CHUNK = 128


def _sc_gather(word_emb, idx3):
    n_chunks = idx3.shape[1]
    n_rows = NW * n_chunks * CHUNK
    per_w = n_chunks * CHUNK
    mesh = plsc.VectorSubcoreMesh(core_axis_name="c", subcore_axis_name="s")

    @functools.partial(
        pl.kernel,
        out_type=jax.ShapeDtypeStruct((n_rows, HIDDEN), jnp.float32),
        mesh=mesh,
        scratch_types=[
            pltpu.VMEM((n_chunks, CHUNK), jnp.int32),
            pltpu.VMEM((CHUNK, HIDDEN), jnp.float32),
            pltpu.SemaphoreType.DMA,
        ],
    )
    def k(table_hbm, idx_hbm, out_hbm, idx_v, rows_v, sem):
        wid = lax.axis_index("s") * NC + lax.axis_index("c")
        base = wid * per_w
        pltpu.sync_copy(idx_hbm.at[wid], idx_v)

        def body(j, _):
            pltpu.async_copy(table_hbm.at[idx_v.at[j]], rows_v, sem).wait()
            pltpu.sync_copy(rows_v, out_hbm.at[pl.ds(base + j * CHUNK, CHUNK)])
            return 0

        lax.fori_loop(0, n_chunks, body, 0)

    return k(word_emb, idx3)


def _tc_ln_body(tt_ref, g_ref, pos_ref, t0_ref, td_ref, gm_ref, bt_ref, o_ref):
    x = g_ref[...]
    t = tt_ref[...].astype(jnp.float32)[..., None]
    x = x + pos_ref[...][None, :, :]
    x = x + t0_ref[...][None, :, :] + t * td_ref[...][None, :, :]
    mean = jnp.mean(x, axis=-1, keepdims=True)
    var = jnp.mean(jnp.square(x - mean), axis=-1, keepdims=True)
    xhat = (x - mean) * lax.rsqrt(var + EPS)
    o_ref[...] = xhat * gm_ref[...][None, :, :] + bt_ref[...][None, :, :]


def kernel(input_ids, token_type_ids, word_emb, pos_emb, type_emb, ln_gamma, ln_beta):
    B, S = input_ids.shape
    H = HIDDEN
    n = B * S
    pad = (-n) % (NW * CHUNK)
    ids_flat = input_ids.reshape(-1).astype(jnp.int32)
    if pad:
        ids_flat = jnp.concatenate([ids_flat, jnp.zeros((pad,), jnp.int32)])
    n_chunks = (n + pad) // (NW * CHUNK)
    idx3 = ids_flat.reshape(NW, n_chunks, CHUNK)

    gathered = _sc_gather(word_emb, idx3)[:n].reshape(B, S, H)

    pos = pos_emb[:S]
    t0 = type_emb[0:1]
    td = (type_emb[1] - type_emb[0])[None, :]
    gm = ln_gamma[None, :]
    bt = ln_beta[None, :]

    BB = 8
    grid = (B // BB,)
    out = pl.pallas_call(
        _tc_ln_body,
        out_shape=jax.ShapeDtypeStruct((B, S, H), jnp.float32),
        grid=grid,
        in_specs=[
            pl.BlockSpec((BB, S), lambda i: (i, 0)),
            pl.BlockSpec((BB, S, H), lambda i: (i, 0, 0)),
            pl.BlockSpec((S, H), lambda i: (0, 0)),
            pl.BlockSpec((1, H), lambda i: (0, 0)),
            pl.BlockSpec((1, H), lambda i: (0, 0)),
            pl.BlockSpec((1, H), lambda i: (0, 0)),
            pl.BlockSpec((1, H), lambda i: (0, 0)),
        ],
        out_specs=pl.BlockSpec((BB, S, H), lambda i: (i, 0, 0)),
    )(token_type_ids.astype(jnp.int32), gathered, pos, t0, td, gm, bt)
    return out


# speedup vs baseline: 6.5803x; 1.1567x over previous
import functools

import jax
import jax.numpy as jnp
from jax import lax
from jax.experimental import pallas as pl
from jax.experimental.pallas import tpu as pltpu
from jax.experimental.pallas import tpu_sc as plsc

HIDDEN = 128
EPS = 1e-12

NC, NS = 2, 16
NW = NC * NS
CHUNK = 128


K = 2
GROUP = K * CHUNK


def _sc_gather(word_emb, idx3):
    n_chunks = idx3.shape[1]
    n_rows = NW * n_chunks * CHUNK
    per_w = n_chunks * CHUNK
    n_groups = n_chunks // K
    assert n_chunks % K == 0 and n_groups >= 3
    mesh = plsc.VectorSubcoreMesh(core_axis_name="c", subcore_axis_name="s")

    @functools.partial(
        pl.kernel,
        out_type=jax.ShapeDtypeStruct((n_rows, HIDDEN), jnp.float32),
        mesh=mesh,
        scratch_types=[
            pltpu.VMEM((n_chunks, CHUNK), jnp.int32),
            pltpu.VMEM((2 * GROUP, HIDDEN), jnp.float32),
            pltpu.SemaphoreType.DMA,
            pltpu.SemaphoreType.DMA,
        ],
    )
    def k(table_hbm, idx_hbm, out_hbm, idx_v, rows_v, gsem, wsem):
        wid = lax.axis_index("s") * NC + lax.axis_index("c")
        base = wid * per_w

        pltpu.sync_copy(idx_hbm.at[wid], idx_v)

        def fire_group(g, half):
            for t in range(K):
                pltpu.async_copy(
                    table_hbm.at[idx_v.at[g * K + t]],
                    rows_v.at[pl.ds(half * GROUP + t * CHUNK, CHUNK)],
                    gsem,
                )

        def drain_gathers():
            for _ in range(K):
                pltpu.make_async_copy(
                    table_hbm.at[idx_v.at[0]], rows_v.at[pl.ds(0, CHUNK)], gsem
                ).wait()

        def fire_write(g, half):
            pltpu.async_copy(
                rows_v.at[pl.ds(half * GROUP, GROUP)],
                out_hbm.at[pl.ds(base + g * GROUP, GROUP)],
                wsem,
            )

        def drain_write():
            pltpu.make_async_copy(
                rows_v.at[pl.ds(0, GROUP)], out_hbm.at[pl.ds(base, GROUP)], wsem
            ).wait()

        fire_group(0, 0)
        fire_group(1, 1)
        drain_gathers()
        fire_write(0, 0)

        def body(g, _):
            half = g % 2
            drain_write()
            fire_group(g + 1, 1 - half)
            drain_gathers()
            fire_write(g, half)
            return 0

        lax.fori_loop(1, n_groups - 1, body, 0)

        gl = n_groups - 1
        drain_write()
        drain_gathers()
        fire_write(gl, gl % 2)
        drain_write()

    return k(word_emb, idx3)


def _tc_ln_body(tt_ref, g_ref, pos_ref, t0_ref, td_ref, gm_ref, bt_ref, o_ref):
    x = g_ref[...]
    t = tt_ref[...].astype(jnp.float32)[..., None]
    x = x + pos_ref[...][None, :, :]
    x = x + t0_ref[...][None, :, :] + t * td_ref[...][None, :, :]
    mean = jnp.mean(x, axis=-1, keepdims=True)
    var = jnp.mean(jnp.square(x - mean), axis=-1, keepdims=True)
    xhat = (x - mean) * lax.rsqrt(var + EPS)
    o_ref[...] = xhat * gm_ref[...][None, :, :] + bt_ref[...][None, :, :]


def kernel(input_ids, token_type_ids, word_emb, pos_emb, type_emb, ln_gamma, ln_beta):
    B, S = input_ids.shape
    H = HIDDEN
    n = B * S
    pad = (-n) % (NW * CHUNK)
    ids_flat = input_ids.reshape(-1).astype(jnp.int32)
    if pad:
        ids_flat = jnp.concatenate([ids_flat, jnp.zeros((pad,), jnp.int32)])
    n_chunks = (n + pad) // (NW * CHUNK)
    idx3 = ids_flat.reshape(NW, n_chunks, CHUNK)

    gathered = _sc_gather(word_emb, idx3)[:n].reshape(B, S, H)

    pos = pos_emb[:S]
    t0 = type_emb[0:1]
    td = (type_emb[1] - type_emb[0])[None, :]
    gm = ln_gamma[None, :]
    bt = ln_beta[None, :]

    BB = 8
    grid = (B // BB,)
    out = pl.pallas_call(
        _tc_ln_body,
        out_shape=jax.ShapeDtypeStruct((B, S, H), jnp.float32),
        grid=grid,
        in_specs=[
            pl.BlockSpec((BB, S), lambda i: (i, 0)),
            pl.BlockSpec((BB, S, H), lambda i: (i, 0, 0)),
            pl.BlockSpec((S, H), lambda i: (0, 0)),
            pl.BlockSpec((1, H), lambda i: (0, 0)),
            pl.BlockSpec((1, H), lambda i: (0, 0)),
            pl.BlockSpec((1, H), lambda i: (0, 0)),
            pl.BlockSpec((1, H), lambda i: (0, 0)),
        ],
        out_specs=pl.BlockSpec((BB, S, H), lambda i: (i, 0, 0)),
    )(token_type_ids.astype(jnp.int32), gathered, pos, t0, td, gm, bt)
    return out


# speedup vs baseline: 7.5396x; 1.1458x over previous
import functools

import jax
import jax.numpy as jnp
from jax import lax
from jax.experimental import pallas as pl
from jax.experimental.pallas import tpu as pltpu
from jax.experimental.pallas import tpu_sc as plsc

HIDDEN = 128
EPS = 1e-12

NC, NS = 2, 16
NW = NC * NS
CHUNK = 128


K = 2
GROUP = K * CHUNK


def _sc_gather(word_emb, idx3):
    n_chunks = idx3.shape[1]
    n_rows = NW * n_chunks * CHUNK
    per_w = n_chunks * CHUNK
    n_groups = n_chunks // K
    assert n_chunks % K == 0 and n_groups >= 3
    mesh = plsc.VectorSubcoreMesh(core_axis_name="c", subcore_axis_name="s")

    @functools.partial(
        pl.kernel,
        out_type=jax.ShapeDtypeStruct((n_rows, HIDDEN), jnp.float32),
        mesh=mesh,
        scratch_types=[
            pltpu.VMEM((n_chunks, CHUNK), jnp.int32),
            pltpu.VMEM((2 * GROUP, HIDDEN), jnp.float32),
            pltpu.SemaphoreType.DMA,
            pltpu.SemaphoreType.DMA,
        ],
    )
    def k(table_hbm, idx_hbm, out_hbm, idx_v, rows_v, gsem, wsem):
        wid = lax.axis_index("s") * NC + lax.axis_index("c")
        base = wid * per_w

        pltpu.sync_copy(idx_hbm.at[wid], idx_v)

        def fire_group(g, half):
            for t in range(K):
                pltpu.async_copy(
                    table_hbm.at[idx_v.at[g * K + t]],
                    rows_v.at[pl.ds(half * GROUP + t * CHUNK, CHUNK)],
                    gsem,
                )

        def drain_gathers():
            for _ in range(K):
                pltpu.make_async_copy(
                    table_hbm.at[idx_v.at[0]], rows_v.at[pl.ds(0, CHUNK)], gsem
                ).wait()

        def fire_write(g, half):
            pltpu.async_copy(
                rows_v.at[pl.ds(half * GROUP, GROUP)],
                out_hbm.at[pl.ds(base + g * GROUP, GROUP)],
                wsem,
            )

        def drain_write():
            pltpu.make_async_copy(
                rows_v.at[pl.ds(0, GROUP)], out_hbm.at[pl.ds(base, GROUP)], wsem
            ).wait()

        fire_group(0, 0)
        fire_group(1, 1)
        drain_gathers()
        fire_write(0, 0)

        def body(g, _):
            half = g % 2
            drain_write()
            fire_group(g + 1, 1 - half)
            drain_gathers()
            fire_write(g, half)
            return 0

        lax.fori_loop(1, n_groups - 1, body, 0)

        gl = n_groups - 1
        drain_write()
        drain_gathers()
        fire_write(gl, gl % 2)
        drain_write()

    return k(word_emb, idx3)


def _tc_ln_body(tt_ref, g_ref, pos_ref, t0_ref, td_ref, gm_ref, bt_ref, o_ref):
    bb, s, h = g_ref.shape
    x = g_ref[...]
    t = tt_ref[...].astype(jnp.float32)[..., None]
    x = x + pos_ref[...][None, :, :]
    x = x + t0_ref[...][None, :, :] + t * td_ref[...][None, :, :]
    x2 = x.reshape(bb * s, h)
    ones = jnp.ones((h, 128), jnp.float32)
    inv_h = 1.0 / h
    s1 = lax.dot_general(x2, ones, (((1,), (0,)), ((), ())),
                         preferred_element_type=jnp.float32)[:, :1]
    s2 = lax.dot_general(x2 * x2, ones, (((1,), (0,)), ((), ())),
                         preferred_element_type=jnp.float32)[:, :1]
    mean = s1 * inv_h
    var = s2 * inv_h - mean * mean
    scale = lax.rsqrt(var + EPS)
    xhat = (x2 - mean) * scale
    o_ref[...] = (xhat.reshape(bb, s, h) * gm_ref[...][None, :, :]
                  + bt_ref[...][None, :, :])


def kernel(input_ids, token_type_ids, word_emb, pos_emb, type_emb, ln_gamma, ln_beta):
    B, S = input_ids.shape
    H = HIDDEN
    n = B * S
    pad = (-n) % (NW * CHUNK)
    ids_flat = input_ids.reshape(-1).astype(jnp.int32)
    if pad:
        ids_flat = jnp.concatenate([ids_flat, jnp.zeros((pad,), jnp.int32)])
    n_chunks = (n + pad) // (NW * CHUNK)
    idx3 = ids_flat.reshape(NW, n_chunks, CHUNK)

    gathered = _sc_gather(word_emb, idx3)[:n].reshape(B, S, H)

    pos = pos_emb[:S]
    t0 = type_emb[0:1]
    td = (type_emb[1] - type_emb[0])[None, :]
    gm = ln_gamma[None, :]
    bt = ln_beta[None, :]

    BB = 16
    grid = (B // BB,)
    out = pl.pallas_call(
        _tc_ln_body,
        out_shape=jax.ShapeDtypeStruct((B, S, H), jnp.float32),
        grid=grid,
        in_specs=[
            pl.BlockSpec((BB, S), lambda i: (i, 0)),
            pl.BlockSpec((BB, S, H), lambda i: (i, 0, 0)),
            pl.BlockSpec((S, H), lambda i: (0, 0)),
            pl.BlockSpec((1, H), lambda i: (0, 0)),
            pl.BlockSpec((1, H), lambda i: (0, 0)),
            pl.BlockSpec((1, H), lambda i: (0, 0)),
            pl.BlockSpec((1, H), lambda i: (0, 0)),
        ],
        out_specs=pl.BlockSpec((BB, S, H), lambda i: (i, 0, 0)),
    )(token_type_ids.astype(jnp.int32), gathered, pos, t0, td, gm, bt)
    return out


# speedup vs baseline: 7.5778x; 1.0051x over previous
import functools

import jax
import jax.numpy as jnp
from jax import lax
from jax.experimental import pallas as pl
from jax.experimental.pallas import tpu as pltpu
from jax.experimental.pallas import tpu_sc as plsc

HIDDEN = 128
EPS = 1e-12

NC, NS = 2, 16
NW = NC * NS
CHUNK = 128


K = 2
GROUP = K * CHUNK


def _sc_gather(word_emb, idx3):
    n_chunks = idx3.shape[1]
    n_rows = NW * n_chunks * CHUNK
    per_w = n_chunks * CHUNK
    n_groups = n_chunks // K
    assert n_chunks % K == 0 and n_groups >= 3
    mesh = plsc.VectorSubcoreMesh(core_axis_name="c", subcore_axis_name="s")

    @functools.partial(
        pl.kernel,
        out_type=jax.ShapeDtypeStruct((n_rows, HIDDEN), jnp.float32),
        mesh=mesh,
        scratch_types=[
            pltpu.VMEM((n_chunks, CHUNK), jnp.int32),
            pltpu.VMEM((2 * GROUP, HIDDEN), jnp.float32),
            pltpu.SemaphoreType.DMA,
            pltpu.SemaphoreType.DMA,
        ],
    )
    def k(table_hbm, idx_hbm, out_hbm, idx_v, rows_v, gsem, wsem):
        wid = lax.axis_index("s") * NC + lax.axis_index("c")
        base = wid * per_w

        pltpu.sync_copy(idx_hbm.at[wid], idx_v)

        def fire_group(g, half):
            for t in range(K):
                pltpu.async_copy(
                    table_hbm.at[idx_v.at[g * K + t]],
                    rows_v.at[pl.ds(half * GROUP + t * CHUNK, CHUNK)],
                    gsem,
                )

        def drain_gathers():
            for _ in range(K):
                pltpu.make_async_copy(
                    table_hbm.at[idx_v.at[0]], rows_v.at[pl.ds(0, CHUNK)], gsem
                ).wait()

        def fire_write(g, half):
            pltpu.async_copy(
                rows_v.at[pl.ds(half * GROUP, GROUP)],
                out_hbm.at[pl.ds(base + g * GROUP, GROUP)],
                wsem,
            )

        def drain_write():
            pltpu.make_async_copy(
                rows_v.at[pl.ds(0, GROUP)], out_hbm.at[pl.ds(base, GROUP)], wsem
            ).wait()

        fire_group(0, 0)
        fire_group(1, 1)
        drain_gathers()
        fire_write(0, 0)

        def body(g, _):
            half = g % 2
            drain_write()
            fire_group(g + 1, 1 - half)
            drain_gathers()
            fire_write(g, half)
            return 0

        lax.fori_loop(1, n_groups - 1, body, 0)

        gl = n_groups - 1
        drain_write()
        drain_gathers()
        fire_write(gl, gl % 2)
        drain_write()

    return k(word_emb, idx3)


def _tc_ln_body(tt_ref, g_ref, pos_ref, td_ref, gm_ref, bt_ref, o_ref):
    bb, s, h = g_ref.shape
    x = g_ref[...]
    t = tt_ref[...].astype(jnp.float32)[..., None]
    x = x + pos_ref[...][None, :, :] + t * td_ref[...][None, :, :]
    x2 = x.reshape(bb * s, h)
    ones = jnp.ones((h, 128), jnp.bfloat16)
    inv_h = 1.0 / h
    xb = x2.astype(jnp.bfloat16)
    s1 = lax.dot_general(xb, ones, (((1,), (0,)), ((), ())),
                         preferred_element_type=jnp.float32)[:, :1]
    s2 = lax.dot_general(xb * xb, ones, (((1,), (0,)), ((), ())),
                         preferred_element_type=jnp.float32)[:, :1]
    mean = s1 * inv_h
    var = s2 * inv_h - mean * mean
    scale = lax.rsqrt(var + EPS)
    xhat = (x2 - mean) * scale
    o_ref[...] = (xhat.reshape(bb, s, h) * gm_ref[...][None, :, :]
                  + bt_ref[...][None, :, :])


def kernel(input_ids, token_type_ids, word_emb, pos_emb, type_emb, ln_gamma, ln_beta):
    B, S = input_ids.shape
    H = HIDDEN
    n = B * S
    pad = (-n) % (NW * CHUNK)
    ids_flat = input_ids.reshape(-1).astype(jnp.int32)
    if pad:
        ids_flat = jnp.concatenate([ids_flat, jnp.zeros((pad,), jnp.int32)])
    n_chunks = (n + pad) // (NW * CHUNK)
    idx3 = ids_flat.reshape(NW, n_chunks, CHUNK)

    gathered = _sc_gather(word_emb, idx3)[:n].reshape(B, S, H)

    pos = pos_emb[:S] + type_emb[0][None, :]
    td = (type_emb[1] - type_emb[0])[None, :]
    gm = ln_gamma[None, :]
    bt = ln_beta[None, :]

    BB = 16
    grid = (B // BB,)
    out = pl.pallas_call(
        _tc_ln_body,
        out_shape=jax.ShapeDtypeStruct((B, S, H), jnp.float32),
        grid=grid,
        in_specs=[
            pl.BlockSpec((BB, S), lambda i: (i, 0)),
            pl.BlockSpec((BB, S, H), lambda i: (i, 0, 0)),
            pl.BlockSpec((S, H), lambda i: (0, 0)),
            pl.BlockSpec((1, H), lambda i: (0, 0)),
            pl.BlockSpec((1, H), lambda i: (0, 0)),
            pl.BlockSpec((1, H), lambda i: (0, 0)),
        ],
        out_specs=pl.BlockSpec((BB, S, H), lambda i: (i, 0, 0)),
    )(token_type_ids.astype(jnp.int32), gathered, pos, td, gm, bt)
    return out


# speedup vs baseline: 8.1574x; 1.0765x over previous
import functools

import jax
import jax.numpy as jnp
from jax import lax
from jax.experimental import pallas as pl
from jax.experimental.pallas import tpu as pltpu
from jax.experimental.pallas import tpu_sc as plsc

HIDDEN = 128
EPS = 1e-12

NC, NS = 2, 16
NW = NC * NS
CHUNK = 128


K = 2
GROUP = K * CHUNK


def _sc_gather(word_emb, idx3):
    n_chunks = idx3.shape[1]
    n_rows = NW * n_chunks * CHUNK
    per_w = n_chunks * CHUNK
    n_groups = n_chunks // K
    assert n_chunks % K == 0 and n_groups >= 3
    mesh = plsc.VectorSubcoreMesh(core_axis_name="c", subcore_axis_name="s")

    @functools.partial(
        pl.kernel,
        out_type=jax.ShapeDtypeStruct((n_rows, HIDDEN), jnp.float32),
        mesh=mesh,
        scratch_types=[
            pltpu.VMEM((n_chunks, CHUNK), jnp.int32),
            pltpu.VMEM((2 * GROUP, HIDDEN), jnp.float32),
            pltpu.SemaphoreType.DMA,
            pltpu.SemaphoreType.DMA,
        ],
    )
    def k(table_hbm, idx_hbm, out_hbm, idx_v, rows_v, gsem, wsem):
        wid = lax.axis_index("s") * NC + lax.axis_index("c")
        base = wid * per_w

        pltpu.sync_copy(idx_hbm.at[wid], idx_v)

        def fire_group(g, half):
            for t in range(K):
                pltpu.async_copy(
                    table_hbm.at[idx_v.at[g * K + t]],
                    rows_v.at[pl.ds(half * GROUP + t * CHUNK, CHUNK)],
                    gsem,
                )

        def drain_gathers():
            for _ in range(K):
                pltpu.make_async_copy(
                    table_hbm.at[idx_v.at[0]], rows_v.at[pl.ds(0, CHUNK)], gsem
                ).wait()

        def fire_write(g, half):
            pltpu.async_copy(
                rows_v.at[pl.ds(half * GROUP, GROUP)],
                out_hbm.at[pl.ds(base + g * GROUP, GROUP)],
                wsem,
            )

        def drain_write():
            pltpu.make_async_copy(
                rows_v.at[pl.ds(0, GROUP)], out_hbm.at[pl.ds(base, GROUP)], wsem
            ).wait()

        fire_group(0, 0)
        fire_group(1, 1)
        drain_gathers()
        fire_write(0, 0)

        def body(g, _):
            half = g % 2
            drain_write()
            fire_group(g + 1, 1 - half)
            drain_gathers()
            fire_write(g, half)
            return 0

        lax.fori_loop(1, n_groups - 1, body, 0)

        gl = n_groups - 1
        drain_write()
        drain_gathers()
        fire_write(gl, gl % 2)
        drain_write()

    return k(word_emb, idx3)


def _tc_ln_body(tt_ref, g_ref, pos_ref, td_ref, gm_ref, bt_ref, o_ref):
    bb, s, h = g_ref.shape
    x = g_ref[...]
    t = tt_ref[...].astype(jnp.float32)[..., None]
    x = x + pos_ref[...][None, :, :] + t * td_ref[...][None, :, :]
    x2 = x.reshape(bb * s, h)
    ones = jnp.ones((h, h), jnp.bfloat16)
    inv_h = 1.0 / h
    xb = x2.astype(jnp.bfloat16)
    s1 = lax.dot_general(xb, ones, (((1,), (0,)), ((), ())),
                         preferred_element_type=jnp.float32)
    s2 = lax.dot_general(xb * xb, ones, (((1,), (0,)), ((), ())),
                         preferred_element_type=jnp.float32)
    mean = s1 * inv_h
    var = s2 * inv_h - mean * mean
    scale = lax.rsqrt(var + EPS) * gm_ref[...].reshape(1, h)
    o_ref[...] = ((x2 - mean) * scale).reshape(bb, s, h) + bt_ref[...][None, :, :]


def kernel(input_ids, token_type_ids, word_emb, pos_emb, type_emb, ln_gamma, ln_beta):
    B, S = input_ids.shape
    H = HIDDEN
    n = B * S
    pad = (-n) % (NW * CHUNK)
    ids_flat = input_ids.reshape(-1).astype(jnp.int32)
    if pad:
        ids_flat = jnp.concatenate([ids_flat, jnp.zeros((pad,), jnp.int32)])
    n_chunks = (n + pad) // (NW * CHUNK)
    idx3 = ids_flat.reshape(NW, n_chunks, CHUNK)

    gathered = _sc_gather(word_emb, idx3)[:n].reshape(B, S, H)

    pos = pos_emb[:S] + type_emb[0][None, :]
    td = (type_emb[1] - type_emb[0])[None, :]
    gm = ln_gamma[None, :]
    bt = ln_beta[None, :]

    BB = 16
    grid = (B // BB,)
    out = pl.pallas_call(
        _tc_ln_body,
        out_shape=jax.ShapeDtypeStruct((B, S, H), jnp.float32),
        grid=grid,
        in_specs=[
            pl.BlockSpec((BB, S), lambda i: (i, 0)),
            pl.BlockSpec((BB, S, H), lambda i: (i, 0, 0)),
            pl.BlockSpec((S, H), lambda i: (0, 0)),
            pl.BlockSpec((1, H), lambda i: (0, 0)),
            pl.BlockSpec((1, H), lambda i: (0, 0)),
            pl.BlockSpec((1, H), lambda i: (0, 0)),
        ],
        out_specs=pl.BlockSpec((BB, S, H), lambda i: (i, 0, 0)),
    )(token_type_ids.astype(jnp.int32), gathered, pos, td, gm, bt)
    return out


# speedup vs baseline: 9.0842x; 1.1136x over previous
import functools

import jax
import jax.numpy as jnp
from jax import lax
from jax.experimental import pallas as pl
from jax.experimental.pallas import tpu as pltpu
from jax.experimental.pallas import tpu_sc as plsc

HIDDEN = 128
EPS = 1e-12

NC, NS = 2, 16
NW = NC * NS
CHUNK = 128


K = 2
GROUP = K * CHUNK


def _sc_gather(word_emb, idx3):
    n_chunks = idx3.shape[1]
    n_rows = NW * n_chunks * CHUNK
    per_w = n_chunks * CHUNK
    n_groups = n_chunks // K
    assert n_chunks % K == 0 and n_groups >= 3
    mesh = plsc.VectorSubcoreMesh(core_axis_name="c", subcore_axis_name="s")

    @functools.partial(
        pl.kernel,
        out_type=jax.ShapeDtypeStruct((n_rows, HIDDEN), jnp.float32),
        mesh=mesh,
        scratch_types=[
            pltpu.VMEM((n_chunks, CHUNK), jnp.int32),
            pltpu.VMEM((2 * GROUP, HIDDEN), jnp.float32),
            pltpu.SemaphoreType.DMA,
            pltpu.SemaphoreType.DMA,
        ],
    )
    def k(table_hbm, idx_hbm, out_hbm, idx_v, rows_v, gsem, wsem):
        wid = lax.axis_index("s") * NC + lax.axis_index("c")
        base = wid * per_w

        pltpu.sync_copy(idx_hbm.at[wid], idx_v)

        def fire_group(g, half):
            for t in range(K):
                pltpu.async_copy(
                    table_hbm.at[idx_v.at[g * K + t]],
                    rows_v.at[pl.ds(half * GROUP + t * CHUNK, CHUNK)],
                    gsem,
                )

        def drain_gathers():
            for _ in range(K):
                pltpu.make_async_copy(
                    table_hbm.at[idx_v.at[0]], rows_v.at[pl.ds(0, CHUNK)], gsem
                ).wait()

        def fire_write(g, half):
            pltpu.async_copy(
                rows_v.at[pl.ds(half * GROUP, GROUP)],
                out_hbm.at[pl.ds(base + g * GROUP, GROUP)],
                wsem,
            )

        def drain_write():
            pltpu.make_async_copy(
                rows_v.at[pl.ds(0, GROUP)], out_hbm.at[pl.ds(base, GROUP)], wsem
            ).wait()

        fire_group(0, 0)
        fire_group(1, 1)
        drain_gathers()
        fire_write(0, 0)

        def body(g, _):
            half = g % 2
            drain_write()
            fire_group(g + 1, 1 - half)
            drain_gathers()
            fire_write(g, half)
            return 0

        lax.fori_loop(1, n_groups - 1, body, 0)

        gl = n_groups - 1
        drain_write()
        drain_gathers()
        fire_write(gl, gl % 2)
        drain_write()

    return k(word_emb, idx3)


def _tc_ln_body(tt_ref, g_ref, pos_ref, td_ref, gm_ref, bt_ref, o_ref):
    bb, s, h = g_ref.shape
    x = g_ref[...]
    t = tt_ref[...].astype(jnp.float32)[..., None]
    x = x + pos_ref[...][None, :, :] + t * td_ref[...][None, :, :]
    x2 = x.reshape(bb * s, h)
    ones = jnp.ones((h, h), jnp.bfloat16)
    inv_h = 1.0 / h
    xb = x2.astype(jnp.bfloat16)
    s1 = lax.dot_general(xb, ones, (((1,), (0,)), ((), ())),
                         preferred_element_type=jnp.float32)
    s2 = lax.dot_general(xb * xb, ones, (((1,), (0,)), ((), ())),
                         preferred_element_type=jnp.float32)
    mean = s1 * inv_h
    var = s2 * inv_h - mean * mean
    scale = lax.rsqrt(var + EPS) * gm_ref[...].reshape(1, h)
    o_ref[...] = ((x2 - mean) * scale).reshape(bb, s, h) + bt_ref[...][None, :, :]


def kernel(input_ids, token_type_ids, word_emb, pos_emb, type_emb, ln_gamma, ln_beta):
    B, S = input_ids.shape
    H = HIDDEN
    n = B * S
    pad = (-n) % (NW * CHUNK)
    ids_flat = input_ids.reshape(-1).astype(jnp.int32)
    if pad:
        ids_flat = jnp.concatenate([ids_flat, jnp.zeros((pad,), jnp.int32)])
    n_chunks = (n + pad) // (NW * CHUNK)
    idx3 = ids_flat.reshape(NW, n_chunks, CHUNK)

    gathered = _sc_gather(word_emb, idx3)[:n].reshape(B, S, H)

    pos = pos_emb[:S] + type_emb[0][None, :]
    td = (type_emb[1] - type_emb[0])[None, :]
    gm = ln_gamma[None, :]
    bt = ln_beta[None, :]

    BB = 32
    grid = (B // BB,)
    out = pl.pallas_call(
        _tc_ln_body,
        out_shape=jax.ShapeDtypeStruct((B, S, H), jnp.float32),
        grid=grid,
        in_specs=[
            pl.BlockSpec((BB, S), lambda i: (i, 0)),
            pl.BlockSpec((BB, S, H), lambda i: (i, 0, 0)),
            pl.BlockSpec((S, H), lambda i: (0, 0)),
            pl.BlockSpec((1, H), lambda i: (0, 0)),
            pl.BlockSpec((1, H), lambda i: (0, 0)),
            pl.BlockSpec((1, H), lambda i: (0, 0)),
        ],
        out_specs=pl.BlockSpec((BB, S, H), lambda i: (i, 0, 0)),
    )(token_type_ids.astype(jnp.int32), gathered, pos, td, gm, bt)
    return out


# speedup vs baseline: 9.5566x; 1.0520x over previous
import functools

import jax
import jax.numpy as jnp
from jax import lax
from jax.experimental import pallas as pl
from jax.experimental.pallas import tpu as pltpu
from jax.experimental.pallas import tpu_sc as plsc

HIDDEN = 128
EPS = 1e-12

NC, NS = 2, 16
NW = NC * NS
CHUNK = 128


K = 2
GROUP = K * CHUNK


def _sc_gather(word_emb, idx3):
    n_chunks = idx3.shape[1]
    n_rows = NW * n_chunks * CHUNK
    per_w = n_chunks * CHUNK
    n_groups = n_chunks // K
    assert n_chunks % K == 0 and n_groups >= 3
    mesh = plsc.VectorSubcoreMesh(core_axis_name="c", subcore_axis_name="s")

    @functools.partial(
        pl.kernel,
        out_type=jax.ShapeDtypeStruct((n_rows, HIDDEN), jnp.float32),
        mesh=mesh,
        scratch_types=[
            pltpu.VMEM((n_chunks, CHUNK), jnp.int32),
            pltpu.VMEM((2 * GROUP, HIDDEN), jnp.float32),
            pltpu.SemaphoreType.DMA,
            pltpu.SemaphoreType.DMA,
        ],
    )
    def k(table_hbm, idx_hbm, out_hbm, idx_v, rows_v, gsem, wsem):
        wid = lax.axis_index("s") * NC + lax.axis_index("c")
        base = wid * per_w

        pltpu.sync_copy(idx_hbm.at[wid], idx_v)

        def fire_group(g, half):
            for t in range(K):
                pltpu.async_copy(
                    table_hbm.at[idx_v.at[g * K + t]],
                    rows_v.at[pl.ds(half * GROUP + t * CHUNK, CHUNK)],
                    gsem,
                )

        def drain_gathers():
            for _ in range(K):
                pltpu.make_async_copy(
                    table_hbm.at[idx_v.at[0]], rows_v.at[pl.ds(0, CHUNK)], gsem
                ).wait()

        def fire_write(g, half):
            pltpu.async_copy(
                rows_v.at[pl.ds(half * GROUP, GROUP)],
                out_hbm.at[pl.ds(base + g * GROUP, GROUP)],
                wsem,
            )

        def drain_write():
            pltpu.make_async_copy(
                rows_v.at[pl.ds(0, GROUP)], out_hbm.at[pl.ds(base, GROUP)], wsem
            ).wait()

        fire_group(0, 0)
        fire_group(1, 1)
        drain_gathers()
        fire_write(0, 0)

        def body(g, _):
            half = g % 2
            drain_write()
            fire_group(g + 1, 1 - half)
            drain_gathers()
            fire_write(g, half)
            return 0

        lax.fori_loop(1, n_groups - 1, body, 0)

        gl = n_groups - 1
        drain_write()
        drain_gathers()
        fire_write(gl, gl % 2)
        drain_write()

    return k(word_emb, idx3)


def _tc_ln_body(tt_ref, g_ref, pos_ref, td_ref, gm_ref, bt_ref, o_ref):
    bb, s, h = g_ref.shape
    x = g_ref[...]
    t = tt_ref[...].astype(jnp.float32)[..., None]
    x = x + pos_ref[...][None, :, :] + t * td_ref[...][None, :, :]
    x2 = x.reshape(bb * s, h)
    ones = jnp.ones((h, h), jnp.bfloat16)
    inv_h = 1.0 / h
    xb = x2.astype(jnp.bfloat16)
    s1 = lax.dot_general(xb, ones, (((1,), (0,)), ((), ())),
                         preferred_element_type=jnp.float32)
    s2 = lax.dot_general(xb * xb, ones, (((1,), (0,)), ((), ())),
                         preferred_element_type=jnp.float32)
    mean = s1 * inv_h
    var = s2 * inv_h - mean * mean
    scale = lax.rsqrt(var + EPS) * gm_ref[...].reshape(1, h)
    o_ref[...] = ((x2 - mean) * scale).reshape(bb, s, h) + bt_ref[...][None, :, :]


def kernel(input_ids, token_type_ids, word_emb, pos_emb, type_emb, ln_gamma, ln_beta):
    B, S = input_ids.shape
    H = HIDDEN
    n = B * S
    pad = (-n) % (NW * CHUNK)
    ids_flat = input_ids.reshape(-1).astype(jnp.int32)
    if pad:
        ids_flat = jnp.concatenate([ids_flat, jnp.zeros((pad,), jnp.int32)])
    n_chunks = (n + pad) // (NW * CHUNK)
    idx3 = ids_flat.reshape(NW, n_chunks, CHUNK)

    gathered = _sc_gather(word_emb, idx3)[:n].reshape(B, S, H)

    pos = pos_emb[:S] + type_emb[0][None, :]
    td = (type_emb[1] - type_emb[0])[None, :]
    gm = ln_gamma[None, :]
    bt = ln_beta[None, :]

    BB = 64
    grid = (B // BB,)
    out = pl.pallas_call(
        _tc_ln_body,
        out_shape=jax.ShapeDtypeStruct((B, S, H), jnp.float32),
        grid=grid,
        in_specs=[
            pl.BlockSpec((BB, S), lambda i: (i, 0)),
            pl.BlockSpec((BB, S, H), lambda i: (i, 0, 0)),
            pl.BlockSpec((S, H), lambda i: (0, 0)),
            pl.BlockSpec((1, H), lambda i: (0, 0)),
            pl.BlockSpec((1, H), lambda i: (0, 0)),
            pl.BlockSpec((1, H), lambda i: (0, 0)),
        ],
        out_specs=pl.BlockSpec((BB, S, H), lambda i: (i, 0, 0)),
    )(token_type_ids.astype(jnp.int32), gathered, pos, td, gm, bt)
    return out


# speedup vs baseline: 9.6412x; 1.0088x over previous
import functools

import jax
import jax.numpy as jnp
from jax import lax
from jax.experimental import pallas as pl
from jax.experimental.pallas import tpu as pltpu
from jax.experimental.pallas import tpu_sc as plsc

HIDDEN = 128
EPS = 1e-12

NC, NS = 2, 16
NW = NC * NS
CHUNK = 128


K = 2
GROUP = K * CHUNK


def _sc_gather(word_emb, idx3):
    n_chunks = idx3.shape[1]
    n_rows = NW * n_chunks * CHUNK
    per_w = n_chunks * CHUNK
    n_groups = n_chunks // K
    assert n_chunks % K == 0 and n_groups >= 3
    mesh = plsc.VectorSubcoreMesh(core_axis_name="c", subcore_axis_name="s")

    @functools.partial(
        pl.kernel,
        out_type=jax.ShapeDtypeStruct((n_rows, HIDDEN), jnp.float32),
        mesh=mesh,
        scratch_types=[
            pltpu.VMEM((n_chunks, CHUNK), jnp.int32),
            pltpu.VMEM((2 * GROUP, HIDDEN), jnp.float32),
            pltpu.SemaphoreType.DMA,
            pltpu.SemaphoreType.DMA,
        ],
    )
    def k(table_hbm, idx_hbm, out_hbm, idx_v, rows_v, gsem, wsem):
        wid = lax.axis_index("s") * NC + lax.axis_index("c")
        base = wid * per_w

        pltpu.sync_copy(idx_hbm.at[wid], idx_v)

        def fire_group(g, half):
            for t in range(K):
                pltpu.async_copy(
                    table_hbm.at[idx_v.at[g * K + t]],
                    rows_v.at[pl.ds(half * GROUP + t * CHUNK, CHUNK)],
                    gsem,
                )

        def drain_gathers():
            for _ in range(K):
                pltpu.make_async_copy(
                    table_hbm.at[idx_v.at[0]], rows_v.at[pl.ds(0, CHUNK)], gsem
                ).wait()

        def fire_write(g, half):
            pltpu.async_copy(
                rows_v.at[pl.ds(half * GROUP, GROUP)],
                out_hbm.at[pl.ds(base + g * GROUP, GROUP)],
                wsem,
            )

        def drain_write():
            pltpu.make_async_copy(
                rows_v.at[pl.ds(0, GROUP)], out_hbm.at[pl.ds(base, GROUP)], wsem
            ).wait()

        fire_group(0, 0)
        fire_group(1, 1)
        drain_gathers()
        fire_write(0, 0)

        def body(g, _):
            half = g % 2
            drain_write()
            fire_group(g + 1, 1 - half)
            drain_gathers()
            fire_write(g, half)
            return 0

        lax.fori_loop(1, n_groups - 1, body, 0)

        gl = n_groups - 1
        drain_write()
        drain_gathers()
        fire_write(gl, gl % 2)
        drain_write()

    return k(word_emb, idx3)


def _tc_ln_body(tt_ref, g_ref, pos_ref, td_ref, gm_ref, bt_ref, o_ref):
    bb, s, h = g_ref.shape
    x = g_ref[...]
    t = tt_ref[...].astype(jnp.float32)[..., None]
    x = x + pos_ref[...][None, :, :] + t * td_ref[...][None, :, :]
    x2 = x.reshape(bb * s, h)
    ones = jnp.ones((h, h), jnp.bfloat16)
    inv_h = 1.0 / h
    xb = x2.astype(jnp.bfloat16)
    s1 = lax.dot_general(xb, ones, (((1,), (0,)), ((), ())),
                         preferred_element_type=jnp.float32)
    s2 = lax.dot_general(xb * xb, ones, (((1,), (0,)), ((), ())),
                         preferred_element_type=jnp.float32)
    mean = s1 * inv_h
    var = s2 * inv_h - mean * mean
    scale = lax.rsqrt(var + EPS) * gm_ref[...].reshape(1, h)
    o_ref[...] = ((x2 - mean) * scale).reshape(bb, s, h) + bt_ref[...][None, :, :]


def kernel(input_ids, token_type_ids, word_emb, pos_emb, type_emb, ln_gamma, ln_beta):
    B, S = input_ids.shape
    H = HIDDEN
    n = B * S
    pad = (-n) % (NW * CHUNK)
    ids_flat = input_ids.reshape(-1).astype(jnp.int32)
    if pad:
        ids_flat = jnp.concatenate([ids_flat, jnp.zeros((pad,), jnp.int32)])
    n_chunks = (n + pad) // (NW * CHUNK)
    idx3 = ids_flat.reshape(NW, n_chunks, CHUNK)

    gathered = _sc_gather(word_emb, idx3)[:n].reshape(B, S, H)

    pos = pos_emb[:S] + type_emb[0][None, :]
    td = (type_emb[1] - type_emb[0])[None, :]
    gm = ln_gamma[None, :]
    bt = ln_beta[None, :]

    BB = 128
    grid = (B // BB,)
    out = pl.pallas_call(
        _tc_ln_body,
        out_shape=jax.ShapeDtypeStruct((B, S, H), jnp.float32),
        grid=grid,
        in_specs=[
            pl.BlockSpec((BB, S), lambda i: (i, 0)),
            pl.BlockSpec((BB, S, H), lambda i: (i, 0, 0)),
            pl.BlockSpec((S, H), lambda i: (0, 0)),
            pl.BlockSpec((1, H), lambda i: (0, 0)),
            pl.BlockSpec((1, H), lambda i: (0, 0)),
            pl.BlockSpec((1, H), lambda i: (0, 0)),
        ],
        out_specs=pl.BlockSpec((BB, S, H), lambda i: (i, 0, 0)),
    )(token_type_ids.astype(jnp.int32), gathered, pos, td, gm, bt)
    return out
